# Initial kernel scaffold; baseline (speedup 1.0000x reference)
import jax
import jax.numpy as jnp
from jax.experimental import pallas as pl


def kernel(rel_embeddings, seq_len):
    raise NotImplementedError("write your pallas kernel here")


# speedup vs baseline: 3.0790x; 3.0790x over previous
import functools

import jax
import jax.numpy as jnp
from jax import lax
from jax.experimental import pallas as pl
from jax.experimental.pallas import tpu as pltpu
from jax.experimental.pallas import tpu_sc as plsc

MAX_DISTANCE = 128
NUM_HEADS = 12
EMBEDDING_DIM = 32
SEQ_LEN = 512

_ROWS = 2 * MAX_DISTANCE + 1
_D = NUM_HEADS * EMBEDDING_DIM
_ROWS_PER_W = 16
_CHUNK_J = 128
_WIN = _ROWS_PER_W + _CHUNK_J


def _make_sc_call():
    info = plsc.get_sparse_core_info()
    nc, ns = info.num_cores, info.num_subcores
    mesh = plsc.VectorSubcoreMesh(core_axis_name="c", subcore_axis_name="s")

    @functools.partial(
        pl.kernel,
        mesh=mesh,
        compiler_params=pltpu.CompilerParams(use_tc_tiling_on_sc=False),
        out_type=jax.ShapeDtypeStruct((SEQ_LEN, SEQ_LEN, _D), jnp.float32),
        scratch_types=[
            pltpu.VMEM((_CHUNK_J,), jnp.int32),
            pltpu.VMEM((16,), jnp.int32),
            pltpu.VMEM((_WIN, _D), jnp.float32),
            pltpu.SemaphoreType.DMA,
            pltpu.SemaphoreType.DMA,
        ],
    )
    def call(table, out, idxa, idxb, buf, gsem, wsem):
        wid = lax.axis_index("s") * nc + lax.axis_index("c")
        i0 = wid * _ROWS_PER_W
        iota = lax.iota(jnp.int32, 16)
        for c in range(4):
            j0 = c * _CHUNK_J
            base = 143 + i0 - j0
            for s in range(8):
                idxa[pl.ds(s * 16, 16)] = jnp.clip(base - s * 16 - iota, 0, _ROWS - 1)
            idxb[...] = jnp.clip(base - 128 - iota, 0, _ROWS - 1)
            ga = pltpu.async_copy(table.at[idxa], buf.at[pl.ds(0, _CHUNK_J)], gsem)
            gb = pltpu.async_copy(table.at[idxb], buf.at[pl.ds(_CHUNK_J, 16)], gsem)
            ga.wait()
            gb.wait()
            writes = []
            for r in range(_ROWS_PER_W):
                writes.append(
                    pltpu.async_copy(
                        buf.at[pl.ds(_ROWS_PER_W - 1 - r, _CHUNK_J)],
                        out.at[i0 + r, pl.ds(j0, _CHUNK_J)],
                        wsem,
                    )
                )
            for w in writes:
                w.wait()

    return call


def kernel(rel_embeddings, seq_len):
    del seq_len
    table = rel_embeddings.reshape(_ROWS, _D)
    out = _make_sc_call()(table)
    return out.reshape(SEQ_LEN, SEQ_LEN, NUM_HEADS, EMBEDDING_DIM)


# speedup vs baseline: 3.0919x; 1.0042x over previous
import functools

import jax
import jax.numpy as jnp
from jax import lax
from jax.experimental import pallas as pl
from jax.experimental.pallas import tpu as pltpu
from jax.experimental.pallas import tpu_sc as plsc

MAX_DISTANCE = 128
NUM_HEADS = 12
EMBEDDING_DIM = 32
SEQ_LEN = 512

_ROWS = 2 * MAX_DISTANCE + 1
_D = NUM_HEADS * EMBEDDING_DIM
_ROWS_PER_W = 16
_CHUNK_J = 128
_WIN = _ROWS_PER_W + _CHUNK_J


def _make_sc_call():
    info = plsc.get_sparse_core_info()
    nc, ns = info.num_cores, info.num_subcores
    mesh = plsc.VectorSubcoreMesh(core_axis_name="c", subcore_axis_name="s")

    @functools.partial(
        pl.kernel,
        mesh=mesh,
        compiler_params=pltpu.CompilerParams(use_tc_tiling_on_sc=False),
        out_type=jax.ShapeDtypeStruct((SEQ_LEN, SEQ_LEN, _D), jnp.float32),
        scratch_types=[
            pltpu.VMEM((_CHUNK_J,), jnp.int32),
            pltpu.VMEM((16,), jnp.int32),
            pltpu.VMEM((2, _WIN, _D), jnp.float32),
            pltpu.SemaphoreType.DMA,
            pltpu.SemaphoreType.DMA,
        ],
    )
    def call(table, out, idxa, idxb, buf, gsem, wsem):
        wid = lax.axis_index("s") * nc + lax.axis_index("c")
        i0 = wid * _ROWS_PER_W
        iota = lax.iota(jnp.int32, 16)

        def fire_gather(c):
            base = 143 + i0 - c * _CHUNK_J
            for s in range(8):
                idxa[pl.ds(s * 16, 16)] = jnp.clip(base - s * 16 - iota, 0, _ROWS - 1)
            idxb[...] = jnp.clip(base - 128 - iota, 0, _ROWS - 1)
            dst = buf.at[c % 2]
            return (
                pltpu.async_copy(table.at[idxa], dst.at[pl.ds(0, _CHUNK_J)], gsem),
                pltpu.async_copy(table.at[idxb], dst.at[pl.ds(_CHUNK_J, 16)], gsem),
            )

        gs = fire_gather(0)
        for c in range(4):
            gs[0].wait()
            gs[1].wait()
            src = buf.at[c % 2]
            writes = [
                pltpu.async_copy(
                    src.at[pl.ds(_ROWS_PER_W - 1 - r, _CHUNK_J)],
                    out.at[i0 + r, pl.ds(c * _CHUNK_J, _CHUNK_J)],
                    wsem,
                )
                for r in range(_ROWS_PER_W)
            ]
            if c < 3:
                gs = fire_gather(c + 1)
            for w in writes:
                w.wait()

    return call


def kernel(rel_embeddings, seq_len):
    del seq_len
    table = rel_embeddings.reshape(_ROWS, _D)
    out = _make_sc_call()(table)
    return out.reshape(SEQ_LEN, SEQ_LEN, NUM_HEADS, EMBEDDING_DIM)
